# baseline (speedup 1.0000x reference)
import functools

import jax
import jax.numpy as jnp
from jax import lax
from jax.experimental import pallas as pl
from jax.experimental.pallas import tpu as pltpu
from jax.experimental.pallas import tpu_sc as plsc

N = 320000
D = 128
H = 64
M = 10000

NC = 2
NS = 16
NW = NC * NS
LANES = 16

M_PAD = 10240
BN = 16384

SLABS = ((0, 229376), (229376, 90624))


def _mlp_body(x_ref, w1_ref, b1_ref, w2_ref, b2_ref, y_ref):
    x = x_ref[...]
    h = jnp.dot(x, w1_ref[...], preferred_element_type=jnp.float32)
    h = h + b1_ref[...]
    h = h * jax.nn.sigmoid(h)
    y2d = lax.dot_general(
        w2_ref[...], h, (((1,), (1,)), ((), ())),
        preferred_element_type=jnp.float32,
    )
    y_ref[...] = y2d[0] + b2_ref[0]


def _make_mlp(start, size):
    first = start // BN

    def call(x, W1, b1, w2row, b2):
        return pl.pallas_call(
            _mlp_body,
            grid=(pl.cdiv(size, BN),),
            in_specs=[
                pl.BlockSpec((BN, D), lambda i: (first + i, 0)),
                pl.BlockSpec((D, H), lambda i: (0, 0)),
                pl.BlockSpec((H,), lambda i: (0,)),
                pl.BlockSpec((1, H), lambda i: (0, 0)),
                pl.BlockSpec(memory_space=pltpu.SMEM),
            ],
            out_specs=pl.BlockSpec((BN,), lambda i: (i,)),
            out_shape=jax.ShapeDtypeStruct((size,), jnp.float32),
        )(x, W1, b1, w2row, b2)

    return call


def _make_seg(start, size):
    chunk = size // NW
    vecs = chunk // LANES

    unroll = next(u for u in (8, 7, 6, 5, 4, 3, 2, 1) if vecs % u == 0)

    def body(y_hbm, idx_hbm, part_hbm, idx_v, y_v, acc_v):
        wid = lax.axis_index("s") * NC + lax.axis_index("c")
        base = wid * chunk

        pltpu.sync_copy(idx_hbm.at[pl.ds(start + base, chunk)],
                        idx_v.at[pl.ds(LANES, chunk)])
        pltpu.sync_copy(y_hbm.at[pl.ds(base, chunk)], y_v)

        def zero_body(i, c):
            for u in range(4):
                acc_v[pl.ds((i * 4 + u) * LANES, LANES)] = jnp.zeros(
                    (LANES,), jnp.float32)
            return c

        lax.fori_loop(0, M_PAD // LANES // 4, zero_body, 0)

        lanes = lax.iota(jnp.int32, LANES)
        m_head = lanes == 0
        m_tail = lanes == LANES - 1

        def seg(t, c):
            for u in range(unroll):
                b = LANES + (t * unroll + u) * LANES
                iv = idx_v[pl.ds(b, LANES)]
                prev = idx_v[pl.ds(b - 1, LANES)]
                nxt = idx_v[pl.ds(b + 1, LANES)]
                yv = y_v[pl.ds(b - LANES, LANES)]
                cs = plsc.cumsum(yv)
                cex = cs - yv
                plsc.addupdate_scatter(
                    acc_v, [iv], cs, mask=m_tail | (iv != nxt))
                plsc.addupdate_scatter(
                    acc_v, [iv], -cex, mask=m_head | (iv != prev))
            return c

        lax.fori_loop(0, vecs // unroll, seg, 0)

        pltpu.sync_copy(acc_v, part_hbm.at[wid])

    return functools.partial(
        pl.kernel,
        out_type=jax.ShapeDtypeStruct((NW, M_PAD), jnp.float32),
        mesh=plsc.VectorSubcoreMesh(core_axis_name="c", subcore_axis_name="s"),
        compiler_params=pltpu.CompilerParams(needs_layout_passes=False),
        scratch_types=[
            pltpu.VMEM((chunk + 2 * LANES,), jnp.int32),
            pltpu.VMEM((chunk,), jnp.float32),
            pltpu.VMEM((M_PAD,), jnp.float32),
        ],
    )(body)


_MLPS = tuple(_make_mlp(s, z) for s, z in SLABS)
_SEGS = tuple(_make_seg(s, z) for s, z in SLABS)


RBN = 2048


def _reduce_body(p1_ref, p2_ref, o_ref):
    o_ref[...] = jnp.sum(p1_ref[...], axis=0) + jnp.sum(p2_ref[...], axis=0)


def _reduce(p1, p2):
    return pl.pallas_call(
        _reduce_body,
        grid=(M_PAD // RBN,),
        in_specs=[
            pl.BlockSpec((NW, RBN), lambda i: (0, i)),
            pl.BlockSpec((NW, RBN), lambda i: (0, i)),
        ],
        out_specs=pl.BlockSpec((RBN,), lambda i: (i,)),
        out_shape=jax.ShapeDtypeStruct((M,), jnp.float32),
    )(p1, p2)


def kernel(scalar_representation, idx_m, W1, b1, W2, b2):
    idx = idx_m.astype(jnp.int32)
    w2row = W2.reshape(1, H)
    ys = [mlp(scalar_representation, W1, b1, w2row, b2) for mlp in _MLPS]
    parts = [seg(y, idx) for seg, y in zip(_SEGS, ys)]
    return _reduce(*parts)


# speedup vs baseline: 1.0152x; 1.0152x over previous
import functools

import jax
import jax.numpy as jnp
from jax import lax
from jax.experimental import pallas as pl
from jax.experimental.pallas import tpu as pltpu
from jax.experimental.pallas import tpu_sc as plsc

N = 320000
D = 128
H = 64
M = 10000

NC = 2
NS = 16
NW = NC * NS
LANES = 16

M_PAD = 10240
BN = 16384

SLABS = ((0, 229376), (229376, 90624))


def _mlp_body(x_ref, w1_ref, b1_ref, w2_ref, b2_ref, y_ref):
    x = x_ref[...]
    h = jnp.dot(x, w1_ref[...], preferred_element_type=jnp.float32)
    h = h + b1_ref[...]
    h = h * jax.nn.sigmoid(h)
    y2d = lax.dot_general(
        w2_ref[...], h, (((1,), (1,)), ((), ())),
        preferred_element_type=jnp.float32,
    )
    y_ref[...] = y2d[0] + b2_ref[0]


def _make_mlp(start, size):
    first = start // BN

    def call(x, W1, b1, w2row, b2):
        return pl.pallas_call(
            _mlp_body,
            grid=(pl.cdiv(size, BN),),
            in_specs=[
                pl.BlockSpec((BN, D), lambda i: (first + i, 0)),
                pl.BlockSpec((D, H), lambda i: (0, 0)),
                pl.BlockSpec((H,), lambda i: (0,)),
                pl.BlockSpec((1, H), lambda i: (0, 0)),
                pl.BlockSpec(memory_space=pltpu.SMEM),
            ],
            out_specs=pl.BlockSpec((BN,), lambda i: (i,)),
            out_shape=jax.ShapeDtypeStruct((size,), jnp.float32),
        )(x, W1, b1, w2row, b2)

    return call


def _make_seg(start, size):
    chunk = size // NW
    vecs = chunk // LANES

    unroll = next(u for u in (8, 7, 6, 5, 4, 3, 2, 1) if vecs % u == 0)

    def body(y_hbm, idx_hbm, part_hbm, idx_v, y_v, acc_v):
        wid = lax.axis_index("s") * NC + lax.axis_index("c")
        base = wid * chunk

        pltpu.sync_copy(idx_hbm.at[pl.ds(start + base, chunk)],
                        idx_v.at[pl.ds(LANES, chunk)])
        pltpu.sync_copy(y_hbm.at[pl.ds(base, chunk)], y_v)

        def zero_body(i, c):
            for u in range(4):
                acc_v[pl.ds((i * 4 + u) * LANES, LANES)] = jnp.zeros(
                    (LANES,), jnp.float32)
            return c

        lax.fori_loop(0, M_PAD // LANES // 4, zero_body, 0)

        lanes = lax.iota(jnp.int32, LANES)
        m_head = lanes == 0
        m_tail = lanes == LANES - 1

        def seg(t, c):
            for u in range(unroll):
                b = LANES + (t * unroll + u) * LANES
                iv = idx_v[pl.ds(b, LANES)]
                prev = idx_v[pl.ds(b - 1, LANES)]
                nxt = idx_v[pl.ds(b + 1, LANES)]
                yv = y_v[pl.ds(b - LANES, LANES)]
                cs = plsc.cumsum(yv)
                cex = cs - yv
                plsc.addupdate_scatter(
                    acc_v, [iv], cs, mask=m_tail | (iv != nxt))
                plsc.addupdate_scatter(
                    acc_v, [iv], -cex, mask=m_head | (iv != prev))
            return c

        lax.fori_loop(0, vecs // unroll, seg, 0)

        pltpu.sync_copy(acc_v, part_hbm.at[wid])

    return functools.partial(
        pl.kernel,
        out_type=jax.ShapeDtypeStruct((NW, M_PAD), jnp.float32),
        mesh=plsc.VectorSubcoreMesh(core_axis_name="c", subcore_axis_name="s"),
        compiler_params=pltpu.CompilerParams(needs_layout_passes=False),
        scratch_types=[
            pltpu.VMEM((chunk + 2 * LANES,), jnp.int32),
            pltpu.VMEM((chunk,), jnp.float32),
            pltpu.VMEM((M_PAD,), jnp.float32),
        ],
    )(body)


_MLPS = tuple(_make_mlp(s, z) for s, z in SLABS)
_SEGS = tuple(_make_seg(s, z) for s, z in SLABS)


def _reduce_body(p1_ref, p2_ref, o_ref):
    s = jnp.sum(p1_ref[...], axis=0) + jnp.sum(p2_ref[...], axis=0)
    o_ref[...] = s[:M]


def _reduce(p1, p2):
    return pl.pallas_call(
        _reduce_body,
        out_shape=jax.ShapeDtypeStruct((M,), jnp.float32),
    )(p1, p2)


def kernel(scalar_representation, idx_m, W1, b1, W2, b2):
    idx = idx_m.astype(jnp.int32)
    w2row = W2.reshape(1, H)
    ys = [mlp(scalar_representation, W1, b1, w2row, b2) for mlp in _MLPS]
    parts = [seg(y, idx) for seg, y in zip(_SEGS, ys)]
    return _reduce(*parts)


# speedup vs baseline: 1.0381x; 1.0225x over previous
import functools

import jax
import jax.numpy as jnp
from jax import lax
from jax.experimental import pallas as pl
from jax.experimental.pallas import tpu as pltpu
from jax.experimental.pallas import tpu_sc as plsc

N = 320000
D = 128
H = 64
M = 10000

NC = 2
NS = 16
NW = NC * NS
LANES = 16

M_PAD = 10240
BN = 16384

SLABS = ((0, 229376), (229376, 90624))


def _mlp_body(x_ref, w1_ref, b1_ref, w2_ref, b2_ref, y_ref):
    x = x_ref[...]
    h = jnp.dot(x, w1_ref[...], preferred_element_type=jnp.float32)
    h = h + b1_ref[...]
    h = h * jax.nn.sigmoid(h)
    y2d = lax.dot_general(
        w2_ref[...], h, (((1,), (1,)), ((), ())),
        preferred_element_type=jnp.float32,
    )
    y_ref[...] = y2d[0] + b2_ref[0]


def _make_mlp(start, size, bn=BN):
    first = start // bn

    def call(x, W1, b1, w2row, b2):
        return pl.pallas_call(
            _mlp_body,
            grid=(pl.cdiv(size, bn),),
            in_specs=[
                pl.BlockSpec((bn, D), lambda i: (first + i, 0)),
                pl.BlockSpec((D, H), lambda i: (0, 0)),
                pl.BlockSpec((H,), lambda i: (0,)),
                pl.BlockSpec((1, H), lambda i: (0, 0)),
                pl.BlockSpec(memory_space=pltpu.SMEM),
            ],
            out_specs=pl.BlockSpec((bn,), lambda i: (i,)),
            out_shape=jax.ShapeDtypeStruct((size,), jnp.float32),
        )(x, W1, b1, w2row, b2)

    return call


def _make_seg(start, size):
    chunk = size // NW
    vecs = chunk // LANES

    unroll = next(u for u in (8, 7, 6, 5, 4, 3, 2, 1) if vecs % u == 0)

    def body(y_hbm, idx_hbm, part_hbm, idx_v, y_v, acc_v):
        wid = lax.axis_index("s") * NC + lax.axis_index("c")
        base = wid * chunk

        pltpu.sync_copy(idx_hbm.at[pl.ds(start + base, chunk)],
                        idx_v.at[pl.ds(LANES, chunk)])
        pltpu.sync_copy(y_hbm.at[pl.ds(base, chunk)], y_v)

        def zero_body(i, c):
            for u in range(4):
                acc_v[pl.ds((i * 4 + u) * LANES, LANES)] = jnp.zeros(
                    (LANES,), jnp.float32)
            return c

        lax.fori_loop(0, M_PAD // LANES // 4, zero_body, 0)

        lanes = lax.iota(jnp.int32, LANES)
        m_head = lanes == 0
        m_tail = lanes == LANES - 1

        def seg(t, c):
            for u in range(unroll):
                b = LANES + (t * unroll + u) * LANES
                iv = idx_v[pl.ds(b, LANES)]
                prev = idx_v[pl.ds(b - 1, LANES)]
                nxt = idx_v[pl.ds(b + 1, LANES)]
                yv = y_v[pl.ds(b - LANES, LANES)]
                cs = plsc.cumsum(yv)
                cex = cs - yv
                plsc.addupdate_scatter(
                    acc_v, [iv], cs, mask=m_tail | (iv != nxt))
                plsc.addupdate_scatter(
                    acc_v, [iv], -cex, mask=m_head | (iv != prev))
            return c

        lax.fori_loop(0, vecs // unroll, seg, 0)

        pltpu.sync_copy(acc_v, part_hbm.at[wid])

    return functools.partial(
        pl.kernel,
        out_type=jax.ShapeDtypeStruct((NW, M_PAD), jnp.float32),
        mesh=plsc.VectorSubcoreMesh(core_axis_name="c", subcore_axis_name="s"),
        compiler_params=pltpu.CompilerParams(needs_layout_passes=False),
        scratch_types=[
            pltpu.VMEM((chunk + 2 * LANES,), jnp.int32),
            pltpu.VMEM((chunk,), jnp.float32),
            pltpu.VMEM((M_PAD,), jnp.float32),
        ],
    )(body)


_MLPS = (_make_mlp(SLABS[0][0], SLABS[0][1], bn=32768),
         _make_mlp(SLABS[1][0], SLABS[1][1], bn=BN))
_SEGS = tuple(_make_seg(s, z) for s, z in SLABS)


def _reduce_body(p1_ref, p2_ref, o_ref):
    s = jnp.sum(p1_ref[...], axis=0) + jnp.sum(p2_ref[...], axis=0)
    o_ref[...] = s[:M]


def _reduce(p1, p2):
    return pl.pallas_call(
        _reduce_body,
        out_shape=jax.ShapeDtypeStruct((M,), jnp.float32),
    )(p1, p2)


def kernel(scalar_representation, idx_m, W1, b1, W2, b2):
    idx = idx_m.astype(jnp.int32)
    w2row = W2.reshape(1, H)
    ys = [mlp(scalar_representation, W1, b1, w2row, b2) for mlp in _MLPS]
    parts = [seg(y, idx) for seg, y in zip(_SEGS, ys)]
    return _reduce(*parts)
